# baseline (speedup 1.0000x reference)
import functools

import jax
import jax.numpy as jnp
from jax import lax
from jax.experimental import pallas as pl
from jax.experimental.pallas import tpu as pltpu
from jax.experimental.pallas import tpu_sc as plsc

K = 8
T = 128
S = 3 * T
J = K * K * K
ST = 144
NC = 2
NS = 16
NW = NC * NS
RB = 128
NG = J // RB
NCHUNK = S // 16


def _sc_body(nw_pairs, x_hbm, out_hbm, xpv, ob0, ob1, sem0, sem1):
    i16 = lax.broadcasted_iota(jnp.int32, (16,), 0)
    wid = lax.axis_index("s") * NC + lax.axis_index("c")

    zf = (i16 * 0).astype(jnp.float32)
    for k in range(K):
        xpv[pl.ds(k * ST, 16)] = zf
        xpv[pl.ds(k * ST + 136, 16)] = zf

    svmap = []
    shifts = []
    for ci in range(NCHUNK):
        sv = i16 + ci * 16
        svmap.append(7 + sv // 3 + sv % 3)
        if ci < 3:
            shifts.append((sv % 3) * 3)

    handles = []
    bufs = (ob0, ob1)
    sems = (sem0, sem1)
    for pp in range(nw_pairs):
        pair = wid * nw_pairs + pp
        for k in range(K):
            pltpu.sync_copy(
                x_hbm.at[pl.ds(pair * K * T + k * T, T)],
                xpv.at[pl.ds(k * ST + 8, T)],
            )

        for g in range(NG):
            gi = pp * NG + g
            ob = bufs[gi % 2]
            if gi >= 2:
                handles[gi - 2].wait()

            def rbody(jr, _):
                j = g * RB + jr
                j0 = j // 64
                j1 = (j // 8) % 8
                j2 = j % 8
                packed = j1 + j0 * 8 + j2 * 64
                for p in range(3):
                    rowsel = (packed >> shifts[p]) & 7
                    base = rowsel * ST
                    for ci in range(p, NCHUNK, 3):
                        v = plsc.load_gather(xpv, [base + svmap[ci]])
                        ob[pl.ds(jr * S + ci * 16, 16)] = v
                return 0

            lax.fori_loop(0, RB, rbody, 0)
            handles.append(
                pltpu.async_copy(
                    ob,
                    out_hbm.at[pl.ds(pair * J * S + g * RB * S, RB * S)],
                    sems[gi % 2],
                )
            )
    handles[-2].wait()
    handles[-1].wait()


@jax.jit
def kernel(x):
    b, c = x.shape[0], x.shape[1]
    bc = b * c
    nw_pairs = bc // NW
    mesh = plsc.VectorSubcoreMesh(core_axis_name="c", subcore_axis_name="s")
    sck = functools.partial(
        pl.kernel,
        mesh=mesh,
        compiler_params=pltpu.CompilerParams(needs_layout_passes=False),
        out_type=jax.ShapeDtypeStruct((bc * J * S,), jnp.float32),
        scratch_types=[
            pltpu.VMEM((K * ST + 16,), jnp.float32),
            pltpu.VMEM((RB * S,), jnp.float32),
            pltpu.VMEM((RB * S,), jnp.float32),
            pltpu.SemaphoreType.DMA,
            pltpu.SemaphoreType.DMA,
        ],
    )(functools.partial(_sc_body, nw_pairs))
    out = sck(x.reshape(-1))
    return out.reshape(b, c, J, S)


# speedup vs baseline: 1.7738x; 1.7738x over previous
import functools

import jax
import jax.numpy as jnp
from jax import lax
from jax.experimental import pallas as pl
from jax.experimental.pallas import tpu as pltpu
from jax.experimental.pallas import tpu_sc as plsc

K = 8
T = 128
S = 3 * T
J = K * K * K
ST = 144
NC = 2
NS = 16
NW = NC * NS
RB = 128
NG = J // RB
NCHUNK = S // 16


def _sc_body(nw_pairs, x_hbm, out_hbm, xpv, ob0, ob1, sem0, sem1):
    i16 = lax.broadcasted_iota(jnp.int32, (16,), 0)
    wid = lax.axis_index("s") * NC + lax.axis_index("c")

    zf = (i16 * 0).astype(jnp.float32)
    for k in range(K):
        xpv[pl.ds(k * ST, 16)] = zf
        xpv[pl.ds(k * ST + 136, 16)] = zf

    svmap = []
    shifts = []
    for ci in range(NCHUNK):
        sv = i16 + ci * 16
        svmap.append(7 + sv // 3 + sv % 3)
        if ci < 3:
            shifts.append((sv % 3) * 3)

    handles = []
    bufs = (ob0, ob1)
    sems = (sem0, sem1)
    for pp in range(nw_pairs):
        pair = wid * nw_pairs + pp
        for k in range(K):
            pltpu.sync_copy(
                x_hbm.at[pl.ds(pair * K * T + k * T, T)],
                xpv.at[pl.ds(k * ST + 8, T)],
            )

        for g in range(NG):
            gi = pp * NG + g
            ob = bufs[gi % 2]
            if gi >= 2:
                handles[gi - 2].wait()

            @plsc.parallel_loop(0, RB, unroll=2)
            def rbody(jr):
                j = g * RB + jr
                j0 = j // 64
                j1 = (j // 8) % 8
                j2 = j % 8
                packed = j1 + j0 * 8 + j2 * 64
                for p in range(3):
                    rowsel = (packed >> shifts[p]) & 7
                    base = rowsel * ST
                    for ci in range(p, NCHUNK, 3):
                        v = plsc.load_gather(xpv, [base + svmap[ci]])
                        ob[pl.ds(jr * S + ci * 16, 16)] = v


            handles.append(
                pltpu.async_copy(
                    ob,
                    out_hbm.at[pl.ds(pair * J * S + g * RB * S, RB * S)],
                    sems[gi % 2],
                )
            )
    handles[-2].wait()
    handles[-1].wait()


@jax.jit
def kernel(x):
    b, c = x.shape[0], x.shape[1]
    bc = b * c
    nw_pairs = bc // NW
    mesh = plsc.VectorSubcoreMesh(core_axis_name="c", subcore_axis_name="s")
    sck = functools.partial(
        pl.kernel,
        mesh=mesh,
        compiler_params=pltpu.CompilerParams(needs_layout_passes=False),
        out_type=jax.ShapeDtypeStruct((bc * J * S,), jnp.float32),
        scratch_types=[
            pltpu.VMEM((K * ST + 16,), jnp.float32),
            pltpu.VMEM((RB * S,), jnp.float32),
            pltpu.VMEM((RB * S,), jnp.float32),
            pltpu.SemaphoreType.DMA,
            pltpu.SemaphoreType.DMA,
        ],
    )(functools.partial(_sc_body, nw_pairs))
    out = sck(x.reshape(-1))
    return out.reshape(b, c, J, S)


# speedup vs baseline: 1.8100x; 1.0204x over previous
import functools

import jax
import jax.numpy as jnp
from jax import lax
from jax.experimental import pallas as pl
from jax.experimental.pallas import tpu as pltpu
from jax.experimental.pallas import tpu_sc as plsc

K = 8
T = 128
S = 3 * T
J = K * K * K
ST = 144
NC = 2
NS = 16
NW = NC * NS
RB = 128
NG = J // RB
NCHUNK = S // 16


def _sc_body(nw_pairs, x_hbm, out_hbm, xpv, ob0, ob1, sem0, sem1):
    i16 = lax.broadcasted_iota(jnp.int32, (16,), 0)
    wid = lax.axis_index("s") * NC + lax.axis_index("c")

    zf = (i16 * 0).astype(jnp.float32)
    for k in range(K):
        xpv[pl.ds(k * ST, 16)] = zf
        xpv[pl.ds(k * ST + 136, 16)] = zf

    svmap = []
    shifts = []
    for ci in range(NCHUNK):
        sv = i16 + ci * 16
        svmap.append(7 + sv // 3 + sv % 3)
        if ci < 3:
            shifts.append((sv % 3) * 3)

    handles = []
    bufs = (ob0, ob1)
    sems = (sem0, sem1)
    for pp in range(nw_pairs):
        pair = wid * nw_pairs + pp
        for k in range(K):
            pltpu.sync_copy(
                x_hbm.at[pl.ds(pair * K * T + k * T, T)],
                xpv.at[pl.ds(k * ST + 8, T)],
            )

        for g in range(NG):
            gi = pp * NG + g
            ob = bufs[gi % 2]
            if gi >= 2:
                handles[gi - 2].wait()

            @plsc.parallel_loop(0, RB, unroll=4)
            def rbody(jr):
                j = g * RB + jr
                j0 = j // 64
                j1 = (j // 8) % 8
                j2 = j % 8
                packed = j1 + j0 * 8 + j2 * 64
                for p in range(3):
                    rowsel = (packed >> shifts[p]) & 7
                    base = rowsel * ST
                    for ci in range(p, NCHUNK, 3):
                        v = plsc.load_gather(xpv, [base + svmap[ci]])
                        ob[pl.ds(jr * S + ci * 16, 16)] = v


            handles.append(
                pltpu.async_copy(
                    ob,
                    out_hbm.at[pl.ds(pair * J * S + g * RB * S, RB * S)],
                    sems[gi % 2],
                )
            )
    handles[-2].wait()
    handles[-1].wait()


@jax.jit
def kernel(x):
    b, c = x.shape[0], x.shape[1]
    bc = b * c
    nw_pairs = bc // NW
    mesh = plsc.VectorSubcoreMesh(core_axis_name="c", subcore_axis_name="s")
    sck = functools.partial(
        pl.kernel,
        mesh=mesh,
        compiler_params=pltpu.CompilerParams(needs_layout_passes=False),
        out_type=jax.ShapeDtypeStruct((bc * J * S,), jnp.float32),
        scratch_types=[
            pltpu.VMEM((K * ST + 16,), jnp.float32),
            pltpu.VMEM((RB * S,), jnp.float32),
            pltpu.VMEM((RB * S,), jnp.float32),
            pltpu.SemaphoreType.DMA,
            pltpu.SemaphoreType.DMA,
        ],
    )(functools.partial(_sc_body, nw_pairs))
    out = sck(x.reshape(-1))
    return out.reshape(b, c, J, S)


# speedup vs baseline: 1.8359x; 1.0143x over previous
import functools

import jax
import jax.numpy as jnp
from jax import lax
from jax.experimental import pallas as pl
from jax.experimental.pallas import tpu as pltpu
from jax.experimental.pallas import tpu_sc as plsc

K = 8
T = 128
S = 3 * T
J = K * K * K
ST = 144
NC = 2
NS = 16
NW = NC * NS
RB = 128
NG = J // RB
NCHUNK = S // 16


def _sc_body(nw_pairs, x_hbm, out_hbm, xpv, ob0, ob1, sem0, sem1):
    i16 = lax.broadcasted_iota(jnp.int32, (16,), 0)
    wid = lax.axis_index("s") * NC + lax.axis_index("c")

    zf = (i16 * 0).astype(jnp.float32)
    for k in range(K):
        xpv[pl.ds(k * ST, 16)] = zf
        xpv[pl.ds(k * ST + 136, 16)] = zf

    svp = []
    shifts = []
    for p in range(3):
        sv = i16 + p * 16
        svp.append(7 + sv // 3 + sv % 3)
        shifts.append((sv % 3) * 3)

    handles = []
    bufs = (ob0, ob1)
    sems = (sem0, sem1)
    for pp in range(nw_pairs):
        pair = wid * nw_pairs + pp
        for k in range(K):
            pltpu.sync_copy(
                x_hbm.at[pl.ds(pair * K * T + k * T, T)],
                xpv.at[pl.ds(k * ST + 8, T)],
            )

        for g in range(NG):
            gi = pp * NG + g
            ob = bufs[gi % 2]
            if gi >= 2:
                handles[gi - 2].wait()

            @plsc.parallel_loop(0, RB, unroll=4)
            def rbody(jr):
                j = g * RB + jr
                j0 = j // 64
                j1 = (j // 8) % 8
                j2 = j % 8
                packed = j1 + j0 * 8 + j2 * 64
                robase = jr * S
                for p in range(3):
                    rowsel = (packed >> shifts[p]) & 7
                    idx = rowsel * ST + svp[p]
                    for m in range(K):
                        src_ref = xpv.at[pl.ds(16 * m, K * ST + 16 - 16 * m)]
                        v = plsc.load_gather(src_ref, [idx])
                        ob[pl.ds(robase + (3 * m + p) * 16, 16)] = v


            handles.append(
                pltpu.async_copy(
                    ob,
                    out_hbm.at[pl.ds(pair * J * S + g * RB * S, RB * S)],
                    sems[gi % 2],
                )
            )
    handles[-2].wait()
    handles[-1].wait()


@jax.jit
def kernel(x):
    b, c = x.shape[0], x.shape[1]
    bc = b * c
    nw_pairs = bc // NW
    mesh = plsc.VectorSubcoreMesh(core_axis_name="c", subcore_axis_name="s")
    sck = functools.partial(
        pl.kernel,
        mesh=mesh,
        compiler_params=pltpu.CompilerParams(needs_layout_passes=False),
        out_type=jax.ShapeDtypeStruct((bc * J * S,), jnp.float32),
        scratch_types=[
            pltpu.VMEM((K * ST + 16,), jnp.float32),
            pltpu.VMEM((RB * S,), jnp.float32),
            pltpu.VMEM((RB * S,), jnp.float32),
            pltpu.SemaphoreType.DMA,
            pltpu.SemaphoreType.DMA,
        ],
    )(functools.partial(_sc_body, nw_pairs))
    out = sck(x.reshape(-1))
    return out.reshape(b, c, J, S)


# speedup vs baseline: 3.8651x; 2.1053x over previous
import functools

import jax
import jax.numpy as jnp
from jax import lax
from jax.experimental import pallas as pl
from jax.experimental.pallas import tpu as pltpu
from jax.experimental.pallas import tpu_sc as plsc

K = 8
T = 128
S = 3 * T
J = K * K * K
ST = 144
NC = 2
NS = 16
NW = NC * NS
RB = 128
NG = J // RB
NCHUNK = S // 16


def _sc_body(nw_pairs, nchan, x_hbm, out_hbm, xpv, ob0, ob1, sem0, sem1):
    i16 = lax.broadcasted_iota(jnp.int32, (16,), 0)
    wid = lax.axis_index("s") * NC + lax.axis_index("c")

    zf = (i16 * 0).astype(jnp.float32)
    for k in range(K):
        xpv[pl.ds(k * ST, 16)] = zf
        xpv[pl.ds(k * ST + 136, 16)] = zf

    svp = []
    shifts = []
    for p in range(3):
        sv = i16 + p * 16
        svp.append(7 + sv // 3 + sv % 3)
        shifts.append((sv % 3) * 3)

    handles = []
    bufs = (ob0, ob1)
    sems = (sem0, sem1)
    for pp in range(nw_pairs):
        pair = wid * nw_pairs + pp
        for k in range(K):
            pltpu.sync_copy(
                x_hbm.at[pl.ds(pair * K * T + k * T, T)],
                xpv.at[pl.ds(k * ST + 8, T)],
            )

        for g in range(NG):
            gi = pp * NG + g
            ob = bufs[gi % 2]
            if gi >= 2:
                handles[gi - 2].wait()

            @plsc.parallel_loop(0, RB, unroll=4)
            def rbody(jr):
                j = g * RB + jr
                j0 = j // 64
                j1 = (j // 8) % 8
                j2 = j % 8
                packed = j1 + j0 * 8 + j2 * 64
                for p in range(3):
                    rowsel = (packed >> shifts[p]) & 7
                    idx = rowsel * ST + svp[p]
                    for m in range(K):
                        src_ref = xpv.at[pl.ds(16 * m, K * ST + 16 - 16 * m)]
                        v = plsc.load_gather(src_ref, [idx])
                        ob[jr, pl.ds((3 * m + p) * 16, 16)] = v


            handles.append(
                pltpu.async_copy(
                    ob,
                    out_hbm.at[pair // nchan, pair % nchan, pl.ds(g * RB, RB)],
                    sems[gi % 2],
                )
            )
    handles[-2].wait()
    handles[-1].wait()


@jax.jit
def kernel(x):
    b, c = x.shape[0], x.shape[1]
    bc = b * c
    nw_pairs = bc // NW
    mesh = plsc.VectorSubcoreMesh(core_axis_name="c", subcore_axis_name="s")
    sck = functools.partial(
        pl.kernel,
        mesh=mesh,
        compiler_params=pltpu.CompilerParams(needs_layout_passes=False),
        out_type=jax.ShapeDtypeStruct((b, c, J, S), jnp.float32),
        scratch_types=[
            pltpu.VMEM((K * ST + 16,), jnp.float32),
            pltpu.VMEM((RB, S), jnp.float32),
            pltpu.VMEM((RB, S), jnp.float32),
            pltpu.SemaphoreType.DMA,
            pltpu.SemaphoreType.DMA,
        ],
    )(functools.partial(_sc_body, nw_pairs, c))
    return sck(x.reshape(-1))


# speedup vs baseline: 3.8777x; 1.0032x over previous
import functools

import jax
import jax.numpy as jnp
from jax import lax
from jax.experimental import pallas as pl
from jax.experimental.pallas import tpu as pltpu
from jax.experimental.pallas import tpu_sc as plsc

K = 8
T = 128
S = 3 * T
J = K * K * K
ST = 144
NC = 2
NS = 16
NW = NC * NS
RB = 128
NG = J // RB
XLEN = K * ST + 16


def _sc_body(nw_pairs, nchan, x_hbm, out_hbm, xpv, ob0, ob1, sem0, sem1):
    i16 = lax.broadcasted_iota(jnp.int32, (16,), 0)
    wid = lax.axis_index("s") * NC + lax.axis_index("c")

    zf = (i16 * 0).astype(jnp.float32)
    for k in range(K):
        xpv[pl.ds(k * ST, 16)] = zf
        xpv[pl.ds(k * ST + 136, 16)] = zf

    svp = []
    shifts = []
    for p in range(3):
        sv = i16 + p * 16
        svp.append(7 + sv // 3 + sv % 3)
        shifts.append((sv % 3) * 3)

    bufs = (ob0, ob1)
    sems = (sem0, sem1)
    ngroups = nw_pairs * NG

    def group(gi, _):
        g = gi % NG
        pair = wid * nw_pairs + gi // NG
        bi = pair // nchan
        ci2 = pair % nchan

        @pl.when(g == 0)
        def _stage():
            for k in range(K):
                pltpu.sync_copy(
                    x_hbm.at[pl.ds(pair * K * T + k * T, T)],
                    xpv.at[pl.ds(k * ST + 8, T)],
                )

        for parity in range(2):

            @pl.when(gi % 2 == parity)
            def _run(parity=parity):
                ob = bufs[parity]
                sem = sems[parity]

                @pl.when(gi >= 2)
                def _drain():
                    pltpu.make_async_copy(
                        ob, out_hbm.at[0, 0, pl.ds(0, RB)], sem
                    ).wait()

                @plsc.parallel_loop(0, RB, unroll=8)
                def rbody(jr):
                    j = g * RB + jr
                    j0 = j // 64
                    j1 = (j // 8) % 8
                    j2 = j % 8
                    packed = j1 + j0 * 8 + j2 * 64
                    for p in range(3):
                        rowsel = (packed >> shifts[p]) & 7
                        idx = rowsel * ST + svp[p]
                        for m in range(K):
                            src_ref = xpv.at[pl.ds(16 * m, XLEN - 16 * m)]
                            v = plsc.load_gather(src_ref, [idx])
                            ob[jr, pl.ds((3 * m + p) * 16, 16)] = v

                pltpu.async_copy(
                    ob, out_hbm.at[bi, ci2, pl.ds(g * RB, RB)], sem
                )

        return 0

    lax.fori_loop(0, ngroups, group, 0)
    for parity in range(2):
        pltpu.make_async_copy(
            bufs[parity], out_hbm.at[0, 0, pl.ds(0, RB)], sems[parity]
        ).wait()


@jax.jit
def kernel(x):
    b, c = x.shape[0], x.shape[1]
    bc = b * c
    nw_pairs = bc // NW
    mesh = plsc.VectorSubcoreMesh(core_axis_name="c", subcore_axis_name="s")
    sck = functools.partial(
        pl.kernel,
        mesh=mesh,
        compiler_params=pltpu.CompilerParams(needs_layout_passes=False),
        out_type=jax.ShapeDtypeStruct((b, c, J, S), jnp.float32),
        scratch_types=[
            pltpu.VMEM((XLEN,), jnp.float32),
            pltpu.VMEM((RB, S), jnp.float32),
            pltpu.VMEM((RB, S), jnp.float32),
            pltpu.SemaphoreType.DMA,
            pltpu.SemaphoreType.DMA,
        ],
    )(functools.partial(_sc_body, nw_pairs, c))
    return sck(x.reshape(-1))


# speedup vs baseline: 4.4654x; 1.1516x over previous
import functools

import jax
import jax.numpy as jnp
from jax import lax
from jax.experimental import pallas as pl
from jax.experimental.pallas import tpu as pltpu
from jax.experimental.pallas import tpu_sc as plsc

K = 8
T = 128
S = 3 * T
J = K * K * K
ST = 152
NC = 2
NS = 16
NW = NC * NS
RB = 128
NG = J // RB
XLEN = K * ST + 16


def _sc_body(nw_pairs, nchan, x_hbm, out_hbm, xpv, ob0, ob1, sem0, sem1):
    i16 = lax.broadcasted_iota(jnp.int32, (16,), 0)
    wid = lax.axis_index("s") * NC + lax.axis_index("c")

    zf = (i16 * 0).astype(jnp.float32)
    for k in range(K):
        xpv[pl.ds(k * ST, 16)] = zf
        xpv[pl.ds(k * ST + 136, 16)] = zf

    svp = []
    shifts = []
    for p in range(3):
        sv = i16 + p * 16
        svp.append(7 + sv // 3 + sv % 3)
        shifts.append((sv % 3) * 3)

    bufs = (ob0, ob1)
    sems = (sem0, sem1)
    ngroups = nw_pairs * NG

    def group(gi, _):
        g = gi % NG
        pair = wid * nw_pairs + gi // NG
        bi = pair // nchan
        ci2 = pair % nchan

        @pl.when(g == 0)
        def _stage():
            for k in range(K):
                pltpu.sync_copy(
                    x_hbm.at[pl.ds(pair * K * T + k * T, T)],
                    xpv.at[pl.ds(k * ST + 8, T)],
                )

        for parity in range(2):

            @pl.when(gi % 2 == parity)
            def _run(parity=parity):
                ob = bufs[parity]
                sem = sems[parity]

                @pl.when(gi >= 2)
                def _drain():
                    pltpu.make_async_copy(
                        ob, out_hbm.at[0, 0, pl.ds(0, RB)], sem
                    ).wait()

                @plsc.parallel_loop(0, RB, unroll=8)
                def rbody(jr):
                    j = g * RB + jr
                    j0 = j // 64
                    j1 = (j // 8) % 8
                    j2 = j % 8
                    packed = j1 + j0 * 8 + j2 * 64
                    for p in range(3):
                        rowsel = (packed >> shifts[p]) & 7
                        idx = rowsel * ST + svp[p]
                        for m in range(K):
                            src_ref = xpv.at[pl.ds(16 * m, XLEN - 16 * m)]
                            v = plsc.load_gather(src_ref, [idx])
                            ob[jr, pl.ds((3 * m + p) * 16, 16)] = v

                pltpu.async_copy(
                    ob, out_hbm.at[bi, ci2, pl.ds(g * RB, RB)], sem
                )

        return 0

    lax.fori_loop(0, ngroups, group, 0)
    for parity in range(2):
        pltpu.make_async_copy(
            bufs[parity], out_hbm.at[0, 0, pl.ds(0, RB)], sems[parity]
        ).wait()


@jax.jit
def kernel(x):
    b, c = x.shape[0], x.shape[1]
    bc = b * c
    nw_pairs = bc // NW
    mesh = plsc.VectorSubcoreMesh(core_axis_name="c", subcore_axis_name="s")
    sck = functools.partial(
        pl.kernel,
        mesh=mesh,
        compiler_params=pltpu.CompilerParams(needs_layout_passes=False),
        out_type=jax.ShapeDtypeStruct((b, c, J, S), jnp.float32),
        scratch_types=[
            pltpu.VMEM((XLEN,), jnp.float32),
            pltpu.VMEM((RB, S), jnp.float32),
            pltpu.VMEM((RB, S), jnp.float32),
            pltpu.SemaphoreType.DMA,
            pltpu.SemaphoreType.DMA,
        ],
    )(functools.partial(_sc_body, nw_pairs, c))
    return sck(x.reshape(-1))


# speedup vs baseline: 4.8797x; 1.0928x over previous
import functools

import jax
import jax.numpy as jnp
from jax import lax
from jax.experimental import pallas as pl
from jax.experimental.pallas import tpu as pltpu
from jax.experimental.pallas import tpu_sc as plsc

K = 8
T = 128
S = 3 * T
J = K * K * K
ST = 130
NC = 2
NS = 16
NW = NC * NS
RB = 128
NG = J // RB
XLEN = K * ST + 16


def _sc_body(nw_pairs, nchan, x_hbm, out_hbm, xpv, xs, ob0, ob1, sem0, sem1):
    i16 = lax.broadcasted_iota(jnp.int32, (16,), 0)
    wid = lax.axis_index("s") * NC + lax.axis_index("c")

    zf = (i16 * 0).astype(jnp.float32)
    pad_idx = (i16 // 2) * ST + 7 + (i16 % 2) * 129
    plsc.store_scatter(xpv, [pad_idx], zf)

    svp = []
    shifts = []
    for p in range(3):
        sv = i16 + p * 16
        svp.append(7 + sv // 3 + sv % 3)
        shifts.append((sv % 3) * 3)

    bufs = (ob0, ob1)
    sems = (sem0, sem1)
    ngroups = nw_pairs * NG

    def group(gi, _):
        g = gi % NG
        pair = wid * nw_pairs + gi // NG
        bi = pair // nchan
        ci2 = pair % nchan

        @pl.when(g == 0)
        def _stage():
            pltpu.sync_copy(x_hbm.at[pl.ds(pair * K * T, K * T)], xs)
            for k in range(K):
                for c2 in range(T // 16):
                    v = xs[pl.ds(k * T + 16 * c2, 16)]
                    plsc.store_scatter(
                        xpv, [i16 + (k * ST + 8 + 16 * c2)], v
                    )

        for parity in range(2):

            @pl.when(gi % 2 == parity)
            def _run(parity=parity):
                ob = bufs[parity]
                sem = sems[parity]

                @pl.when(gi >= 2)
                def _drain():
                    pltpu.make_async_copy(
                        ob, out_hbm.at[0, 0, pl.ds(0, RB)], sem
                    ).wait()

                @plsc.parallel_loop(0, RB, unroll=8)
                def rbody(jr):
                    j = g * RB + jr
                    j0 = j // 64
                    j1 = (j // 8) % 8
                    j2 = j % 8
                    packed = j1 + j0 * 8 + j2 * 64
                    for p in range(3):
                        rowsel = (packed >> shifts[p]) & 7
                        idx = rowsel * ST + svp[p]
                        for m in range(K):
                            src_ref = xpv.at[pl.ds(16 * m, XLEN - 16 * m)]
                            v = plsc.load_gather(src_ref, [idx])
                            ob[jr, pl.ds((3 * m + p) * 16, 16)] = v

                pltpu.async_copy(
                    ob, out_hbm.at[bi, ci2, pl.ds(g * RB, RB)], sem
                )

        return 0

    lax.fori_loop(0, ngroups, group, 0)
    for parity in range(2):
        pltpu.make_async_copy(
            bufs[parity], out_hbm.at[0, 0, pl.ds(0, RB)], sems[parity]
        ).wait()


@jax.jit
def kernel(x):
    b, c = x.shape[0], x.shape[1]
    bc = b * c
    nw_pairs = bc // NW
    mesh = plsc.VectorSubcoreMesh(core_axis_name="c", subcore_axis_name="s")
    sck = functools.partial(
        pl.kernel,
        mesh=mesh,
        compiler_params=pltpu.CompilerParams(needs_layout_passes=False),
        out_type=jax.ShapeDtypeStruct((b, c, J, S), jnp.float32),
        scratch_types=[
            pltpu.VMEM((XLEN,), jnp.float32),
            pltpu.VMEM((K * T,), jnp.float32),
            pltpu.VMEM((RB, S), jnp.float32),
            pltpu.VMEM((RB, S), jnp.float32),
            pltpu.SemaphoreType.DMA,
            pltpu.SemaphoreType.DMA,
        ],
    )(functools.partial(_sc_body, nw_pairs, c))
    return sck(x.reshape(-1))


# speedup vs baseline: 5.0058x; 1.0259x over previous
import functools

import jax
import jax.numpy as jnp
from jax import lax
from jax.experimental import pallas as pl
from jax.experimental.pallas import tpu as pltpu
from jax.experimental.pallas import tpu_sc as plsc

K = 8
T = 128
S = 3 * T
J = K * K * K
ST = 130
NC = 2
NS = 16
NW = NC * NS
RB = 64
NG = J // RB
XLEN = K * ST + 16


def _sc_body(nw_pairs, nchan, x_hbm, out_hbm, xpv, xs, ob0, ob1, sem0, sem1):
    i16 = lax.broadcasted_iota(jnp.int32, (16,), 0)
    wid = lax.axis_index("s") * NC + lax.axis_index("c")

    zf = (i16 * 0).astype(jnp.float32)
    pad_idx = (i16 // 2) * ST + 7 + (i16 % 2) * 129
    plsc.store_scatter(xpv, [pad_idx], zf)

    svp = []
    shifts = []
    for p in range(3):
        sv = i16 + p * 16
        svp.append(7 + sv // 3 + sv % 3)
        shifts.append((sv % 3) * 3)

    bufs = (ob0, ob1)
    sems = (sem0, sem1)
    ngroups = nw_pairs * NG

    def group(gi, _):
        g = gi % NG
        pair = wid * nw_pairs + gi // NG
        bi = pair // nchan
        ci2 = pair % nchan

        @pl.when(g == 0)
        def _stage():
            pltpu.sync_copy(x_hbm.at[pl.ds(pair * K * T, K * T)], xs)
            for k in range(K):
                for c2 in range(T // 16):
                    v = xs[pl.ds(k * T + 16 * c2, 16)]
                    plsc.store_scatter(
                        xpv, [i16 + (k * ST + 8 + 16 * c2)], v
                    )

        for parity in range(2):

            @pl.when(gi % 2 == parity)
            def _run(parity=parity):
                ob = bufs[parity]
                sem = sems[parity]

                @pl.when(gi >= 2)
                def _drain():
                    pltpu.make_async_copy(
                        ob, out_hbm.at[0, 0, pl.ds(0, RB)], sem
                    ).wait()

                @plsc.parallel_loop(0, RB, unroll=8)
                def rbody(jr):
                    j = g * RB + jr
                    j0 = j // 64
                    j1 = (j // 8) % 8
                    j2 = j % 8
                    packed = j1 + j0 * 8 + j2 * 64
                    for p in range(3):
                        rowsel = (packed >> shifts[p]) & 7
                        idx = rowsel * ST + svp[p]
                        for m in range(K):
                            src_ref = xpv.at[pl.ds(16 * m, XLEN - 16 * m)]
                            v = plsc.load_gather(src_ref, [idx])
                            ob[jr, pl.ds((3 * m + p) * 16, 16)] = v

                pltpu.async_copy(
                    ob, out_hbm.at[bi, ci2, pl.ds(g * RB, RB)], sem
                )

        return 0

    lax.fori_loop(0, ngroups, group, 0)
    for parity in range(2):
        pltpu.make_async_copy(
            bufs[parity], out_hbm.at[0, 0, pl.ds(0, RB)], sems[parity]
        ).wait()


@jax.jit
def kernel(x):
    b, c = x.shape[0], x.shape[1]
    bc = b * c
    nw_pairs = bc // NW
    mesh = plsc.VectorSubcoreMesh(core_axis_name="c", subcore_axis_name="s")
    sck = functools.partial(
        pl.kernel,
        mesh=mesh,
        compiler_params=pltpu.CompilerParams(needs_layout_passes=False),
        out_type=jax.ShapeDtypeStruct((b, c, J, S), jnp.float32),
        scratch_types=[
            pltpu.VMEM((XLEN,), jnp.float32),
            pltpu.VMEM((K * T,), jnp.float32),
            pltpu.VMEM((RB, S), jnp.float32),
            pltpu.VMEM((RB, S), jnp.float32),
            pltpu.SemaphoreType.DMA,
            pltpu.SemaphoreType.DMA,
        ],
    )(functools.partial(_sc_body, nw_pairs, c))
    return sck(x.reshape(-1))
